# Initial kernel scaffold; baseline (speedup 1.0000x reference)
import jax
import jax.numpy as jnp
from jax.experimental import pallas as pl


def kernel(data, indices, axis):
    raise NotImplementedError("write your pallas kernel here")


# speedup vs baseline: 1.0926x; 1.0926x over previous
import functools

import jax
import jax.numpy as jnp
from jax import lax
from jax.experimental import pallas as pl
from jax.experimental.pallas import tpu as pltpu
from jax.experimental.pallas import tpu_sc as plsc

_NC = 2
_NS = 16
_NW = _NC * _NS

_CHUNK = 1024


@functools.partial(jax.jit, static_argnames=("b_per_w",))
def _sc_gather(data, idx_flat, *, b_per_w):
    n_chunks = b_per_w // _CHUNK
    mesh = plsc.VectorSubcoreMesh(
        core_axis_name="c", subcore_axis_name="s",
        num_cores=_NC, num_subcores=_NS,
    )

    @functools.partial(
        pl.kernel,
        out_type=jax.ShapeDtypeStruct((idx_flat.shape[0], data.shape[1]),
                                      data.dtype),
        mesh=mesh,
        scratch_types=[
            pltpu.VMEM((_CHUNK,), jnp.int32),
            pltpu.VMEM((_CHUNK, data.shape[1]), data.dtype),
            pltpu.SemaphoreType.DMA,
        ],
        compiler_params=pltpu.CompilerParams(use_tc_tiling_on_sc=False),
    )
    def k(table_hbm, idx_hbm, out_hbm, idx_v, rows_v, sem):
        wid = lax.axis_index("s") * _NC + lax.axis_index("c")

        def chunk_body(g, carry):
            base = wid * b_per_w + g * _CHUNK
            pltpu.sync_copy(idx_hbm.at[pl.ds(base, _CHUNK)], idx_v)
            pltpu.async_copy(table_hbm.at[idx_v], rows_v, sem).wait()
            pltpu.sync_copy(rows_v, out_hbm.at[pl.ds(base, _CHUNK)])
            return carry

        lax.fori_loop(0, n_chunks, chunk_body, 0)

    return k(data, idx_flat)


def kernel(data, indices, axis):
    del axis
    b_total = indices.shape[0] * indices.shape[1]
    idx_flat = indices.reshape(b_total)
    out = _sc_gather(data, idx_flat, b_per_w=b_total // _NW)
    return out.reshape(indices.shape + data.shape[1:])


# speedup vs baseline: 1.1109x; 1.0167x over previous
import functools

import jax
import jax.numpy as jnp
from jax import lax
from jax.experimental import pallas as pl
from jax.experimental.pallas import tpu as pltpu
from jax.experimental.pallas import tpu_sc as plsc

_NC = 2
_NS = 16
_NW = _NC * _NS

_CHUNK = 512
_NBUF = 5


@functools.partial(jax.jit, static_argnames=("b_per_w",))
def _sc_gather(data, idx_flat, *, b_per_w):
    n_chunks = b_per_w // _CHUNK
    assert n_chunks % _NBUF == 0 and n_chunks >= 2 * _NBUF
    d = data.shape[1]

    mesh = plsc.VectorSubcoreMesh(
        core_axis_name="c", subcore_axis_name="s",
        num_cores=_NC, num_subcores=_NS,
    )

    @functools.partial(
        pl.kernel,
        out_type=jax.ShapeDtypeStruct((idx_flat.shape[0], d), data.dtype),
        mesh=mesh,
        scratch_types=[
            pltpu.VMEM((b_per_w,), jnp.int32),
            pltpu.VMEM((_NBUF, _CHUNK, d), data.dtype),
            pltpu.SemaphoreType.DMA((_NBUF,)),
            pltpu.SemaphoreType.DMA((_NBUF,)),
        ],
        compiler_params=pltpu.CompilerParams(use_tc_tiling_on_sc=False),
    )
    def k(table_hbm, idx_hbm, out_hbm, idx_v, rows_v, gsem, wsem):
        wid = lax.axis_index("s") * _NC + lax.axis_index("c")
        base = wid * b_per_w

        pltpu.sync_copy(idx_hbm.at[pl.ds(base, b_per_w)], idx_v)

        def start_gather(g, b):
            return pltpu.async_copy(
                table_hbm.at[idx_v.at[pl.ds(g * _CHUNK, _CHUNK)]],
                rows_v.at[b], gsem.at[b])

        def start_write(g, b):
            return pltpu.async_copy(
                rows_v.at[b], out_hbm.at[pl.ds(base + g * _CHUNK, _CHUNK)],
                wsem.at[b])

        def wait_gather(b):
            pltpu.make_async_copy(
                table_hbm.at[idx_v.at[pl.ds(0, _CHUNK)]],
                rows_v.at[b], gsem.at[b]).wait()

        def wait_write(b):
            pltpu.make_async_copy(
                rows_v.at[b], out_hbm.at[pl.ds(base, _CHUNK)],
                wsem.at[b]).wait()

        start_gather(0, 0)
        for g in range(1, _NBUF):
            start_gather(g, g)
            wait_gather(g - 1)
            start_write(g - 1, g - 1)

        def outer(t, carry):
            g0 = _NBUF + t * _NBUF
            for b in range(_NBUF):
                g = g0 + b
                wait_write(b)
                start_gather(g, b)
                bp = (b - 1) % _NBUF
                wait_gather(bp)
                start_write(g - 1, bp)
            return carry

        lax.fori_loop(0, n_chunks // _NBUF - 1, outer, 0)

        last_b = (n_chunks - 1) % _NBUF
        wait_gather(last_b)
        start_write(n_chunks - 1, last_b)
        for b in range(_NBUF):
            wait_write(b)

    return k(data, idx_flat)


def kernel(data, indices, axis):
    del axis
    b_total = indices.shape[0] * indices.shape[1]
    idx_flat = indices.reshape(b_total)
    out = _sc_gather(data, idx_flat, b_per_w=b_total // _NW)
    return out.reshape(indices.shape + data.shape[1:])


# speedup vs baseline: 1.9375x; 1.7441x over previous
import functools

import jax
import jax.numpy as jnp
from jax import lax
from jax.experimental import pallas as pl
from jax.experimental.pallas import tpu as pltpu
from jax.experimental.pallas import tpu_sc as plsc

_NC = 2
_NS = 16
_NW = _NC * _NS

_CHUNK = 512
_NBUF = 5


@functools.partial(jax.jit, static_argnames=("b_per_w",))
def _sc_gather(data, idx_flat, *, b_per_w):
    n_chunks = b_per_w // _CHUNK
    assert n_chunks % _NBUF == 0 and n_chunks >= 2 * _NBUF
    d = data.shape[1]

    mesh = plsc.VectorSubcoreMesh(
        core_axis_name="c", subcore_axis_name="s",
        num_cores=_NC, num_subcores=_NS,
    )

    @functools.partial(
        pl.kernel,
        out_type=jax.ShapeDtypeStruct((idx_flat.shape[0], d), data.dtype),
        mesh=mesh,
        scratch_types=[
            pltpu.VMEM((b_per_w,), jnp.int32),
            pltpu.VMEM((_NBUF, _CHUNK, d), data.dtype),
            pltpu.SemaphoreType.DMA((_NBUF,)),
            pltpu.SemaphoreType.DMA((_NBUF,)),
        ],
        compiler_params=pltpu.CompilerParams(use_tc_tiling_on_sc=False),
    )
    def k(table_hbm, idx_hbm, out_hbm, idx_v, rows_v, gsem, wsem):
        wid = lax.axis_index("s") * _NC + lax.axis_index("c")
        base = wid * b_per_w

        pltpu.sync_copy(idx_hbm.at[pl.ds(base, b_per_w)], idx_v)

        def start_gather(g, b):
            return pltpu.async_copy(
                table_hbm.at[idx_v.at[pl.ds(g * _CHUNK, _CHUNK)]],
                rows_v.at[b], gsem.at[b])

        def start_write(g, b):
            return pltpu.async_copy(
                rows_v.at[b], out_hbm.at[pl.ds(base + g * _CHUNK, _CHUNK)],
                wsem.at[b])

        def wait_gather(b):
            pltpu.make_async_copy(
                table_hbm.at[idx_v.at[pl.ds(0, _CHUNK)]],
                rows_v.at[b], gsem.at[b]).wait()

        def wait_write(b):
            pltpu.make_async_copy(
                rows_v.at[b], out_hbm.at[pl.ds(base, _CHUNK)],
                wsem.at[b]).wait()

        start_gather(0, 0)
        for g in range(1, _NBUF):
            start_gather(g, g)
            wait_gather(g - 1)
            start_write(g - 1, g - 1)

        def outer(t, carry):
            g0 = _NBUF + t * _NBUF
            for b in range(_NBUF):
                g = g0 + b
                wait_write(b)
                start_gather(g, b)
                bp = (b - 1) % _NBUF
                wait_gather(bp)
                start_write(g - 1, bp)
            return carry

        lax.fori_loop(0, n_chunks // _NBUF - 1, outer, 0)

        last_b = (n_chunks - 1) % _NBUF
        wait_gather(last_b)
        start_write(n_chunks - 1, last_b)
        for b in range(_NBUF):
            wait_write(b)

    return k(data, idx_flat)


def kernel(data, indices, axis):
    del axis
    b, j = indices.shape
    d = data.shape[1]
    idx_flat = indices.T.reshape(b * j)
    out = _sc_gather(data, idx_flat, b_per_w=(b * j) // _NW)
    return out.reshape(j, b, d).transpose(1, 0, 2)


# speedup vs baseline: 1.9403x; 1.0015x over previous
import functools

import jax
import jax.numpy as jnp
from jax import lax
from jax.experimental import pallas as pl
from jax.experimental.pallas import tpu as pltpu
from jax.experimental.pallas import tpu_sc as plsc

_NC = 2
_NS = 16
_NW = _NC * _NS

_NBUF = 5


@jax.jit
def _sc_gather(data, idx_t):
    n_j, b = idx_t.shape
    d = data.shape[1]
    bw = b // _NW
    assert n_j % _NBUF == 0 and n_j >= 2 * _NBUF
    mesh = plsc.VectorSubcoreMesh(
        core_axis_name="c", subcore_axis_name="s",
        num_cores=_NC, num_subcores=_NS,
    )

    @functools.partial(
        pl.kernel,
        out_type=jax.ShapeDtypeStruct((n_j, b, d), data.dtype),
        mesh=mesh,
        scratch_types=[
            pltpu.VMEM((n_j, bw), jnp.int32),
            pltpu.VMEM((_NBUF, bw, d), data.dtype),
            pltpu.SemaphoreType.DMA,
            pltpu.SemaphoreType.DMA((_NBUF,)),
            pltpu.SemaphoreType.DMA((_NBUF,)),
        ],
        compiler_params=pltpu.CompilerParams(use_tc_tiling_on_sc=False),
    )
    def k(table_hbm, idx_hbm, out_hbm, idx_v, rows_v, isem, gsem, wsem):
        wid = lax.axis_index("s") * _NC + lax.axis_index("c")
        b0 = wid * bw

        pltpu.async_copy(
            idx_hbm.at[:, pl.ds(b0, bw)], idx_v, isem).wait()

        def start_gather(g, slot):
            return pltpu.async_copy(
                table_hbm.at[idx_v.at[g]], rows_v.at[slot], gsem.at[slot])

        def start_write(g, slot):
            return pltpu.async_copy(
                rows_v.at[slot], out_hbm.at[g, pl.ds(b0, bw)],
                wsem.at[slot])

        def wait_gather(slot):
            pltpu.make_async_copy(
                table_hbm.at[idx_v.at[0]], rows_v.at[slot],
                gsem.at[slot]).wait()

        def wait_write(slot):
            pltpu.make_async_copy(
                rows_v.at[slot], out_hbm.at[0, pl.ds(b0, bw)],
                wsem.at[slot]).wait()

        start_gather(0, 0)
        for g in range(1, _NBUF):
            start_gather(g, g)
            wait_gather(g - 1)
            start_write(g - 1, g - 1)

        def outer(t, carry):
            g0 = _NBUF + t * _NBUF
            for s in range(_NBUF):
                g = g0 + s
                wait_write(s)
                start_gather(g, s)
                sp = (s - 1) % _NBUF
                wait_gather(sp)
                start_write(g - 1, sp)
            return carry

        lax.fori_loop(0, n_j // _NBUF - 1, outer, 0)

        last_s = (n_j - 1) % _NBUF
        wait_gather(last_s)
        start_write(n_j - 1, last_s)
        for s in range(_NBUF):
            wait_write(s)

    return k(data, idx_t)


def kernel(data, indices, axis):
    del axis
    out = _sc_gather(data, indices.T)
    return out.transpose(1, 0, 2)
